# baseline (speedup 1.0000x reference)
import jax
import jax.numpy as jnp
from jax.experimental import pallas as pl

B, F, T = 256, 8, 20
N = 20000
D = N + 1


def _id_body(x_ref, o_ref):
    o_ref[:, :] = x_ref[:, :]


def kernel(tags, feature_counts, ln1_g, ln1_b, W1, b1, ln2_g, ln2_b, W2, b2, W3, b3, W4, b4, W5, b5):
    tags2 = tags.reshape(B, F * T)
    pos = jnp.arange(F * T)
    row = pos // T
    eq = tags2[:, :, None] == tags2[:, None, :]
    same_row_before = (row[:, None] == row[None, :]) & (pos[None, :] < pos[:, None])
    dup = (eq & same_row_before[None]).any(-1)
    w = (~dup).astype(jnp.float32)
    c = jnp.einsum('bij,bj->bi', eq.astype(jnp.float32), w)
    K = w.sum(-1)
    S2 = (w * c).sum(-1)
    m = K / N
    var = S2 / N - m * m
    s = jax.lax.rsqrt(var + 1e-5)
    base = (-m * s)[:, None] * ln1_g[None, :] + ln1_b[None, :]
    y = jnp.concatenate([feature_counts[:, None] / 100.0, base], axis=1)
    val = (c - m[:, None]) * s[:, None] * ln1_g[tags2] + ln1_b[tags2]
    y = y.at[jnp.arange(B)[:, None], 1 + tags2].set(val)

    def _gelu(x):
        return jax.nn.gelu(x, approximate=False)

    def _ln(x, g, b2_, eps=1e-5):
        mm = x.mean(-1, keepdims=True)
        vv = ((x - mm) ** 2).mean(-1, keepdims=True)
        return (x - mm) / jnp.sqrt(vv + eps) * g + b2_

    h = _gelu(y @ W1 + b1)
    h = _ln(h, ln2_g, ln2_b)
    h = _gelu(h @ W2 + b2)
    enc = h @ W3 + b3
    d = _gelu(enc @ W4 + b4)
    d = pl.pallas_call(
        _id_body,
        out_shape=jax.ShapeDtypeStruct(d.shape, d.dtype),
    )(d)
    dec = d @ W5 + b5
    return (y, enc, dec)


# speedup vs baseline: 3.2108x; 3.2108x over previous
import functools

import jax
import jax.numpy as jnp
from jax import lax
from jax.experimental import pallas as pl
from jax.experimental.pallas import tpu as pltpu
from jax.experimental.pallas import tpu_sc as plsc

B, F, T = 256, 8, 20
NT = F * T
N = 20000
D = N + 1
H4 = 1024
EPS = 1e-5

NC, NS = 2, 16
NW = NC * NS
SPB = B // NW
GPAD = 20016


def _stats_body(tags_ref, c_ref, ms_ref, w_scr):
    t = tags_ref[:, :]
    dup = jnp.zeros((NT, B), jnp.float32)
    for d in range(1, T):
        eq = (t[d:, :] == t[:-d, :]).astype(jnp.float32)
        eq = jnp.concatenate([jnp.zeros((d, B), jnp.float32), eq], axis=0)
        valid = ((jnp.arange(NT) % T) >= d).astype(jnp.float32)
        dup = jnp.maximum(dup, eq * valid[:, None])
    w = 1.0 - dup
    w_scr[:, :] = w

    def body(k, c):
        tk = tags_ref[pl.ds(k, 1), :]
        wk = w_scr[pl.ds(k, 1), :]
        return c + jnp.where(t == tk, wk, 0.0)

    c = lax.fori_loop(0, NT, body, jnp.zeros((NT, B), jnp.float32))
    ksum = jnp.sum(w, axis=0, keepdims=True)
    s2 = jnp.sum(w * c, axis=0, keepdims=True)
    m = ksum / N
    var = s2 / N - m * m
    s = lax.rsqrt(var + EPS)
    c_ref[:, :] = c
    ms_ref[:, :] = jnp.concatenate(
        [m, s, jnp.zeros((6, B), jnp.float32)], axis=0)


def _stats_call(tags_t):
    return pl.pallas_call(
        _stats_body,
        out_shape=(
            jax.ShapeDtypeStruct((NT, B), jnp.float32),
            jax.ShapeDtypeStruct((8, B), jnp.float32),
        ),
        scratch_shapes=[pltpu.VMEM((NT, B), jnp.float32)],
    )(tags_t)


def _sc_body(gpad_hbm, bpad_hbm, tags_hbm, c_hbm, ms_hbm, y_hbm,
             g_v, b_v, rowa_v, rowb_v, tag_v, c_v, ms_v, sem_in, sem_a, sem_b):
    wid = lax.axis_index("s") * NC + lax.axis_index("c")
    base = wid * SPB
    pltpu.sync_copy(gpad_hbm, g_v)
    pltpu.sync_copy(bpad_hbm, b_v)
    rows = (rowa_v, rowb_v)
    sems = (sem_a, sem_b)
    lane = jnp.arange(16, dtype=jnp.int32)
    pending = [None, None]
    for sl in range(SPB):
        bufi = sl % 2
        rv = rows[bufi]
        if pending[bufi] is not None:
            pending[bufi].wait()
        b = base + sl
        pltpu.sync_copy(tags_hbm.at[b], tag_v)
        pltpu.sync_copy(c_hbm.at[b], c_v)
        pltpu.sync_copy(ms_hbm.at[b], ms_v)
        m = ms_v[0:16]
        s = ms_v[16:32]
        fc = ms_v[32:48]
        a = -(m * s)

        def fill(j, _, rv=rv, a=a):
            o = j * 16
            rv[pl.ds(o, 16)] = a * g_v[pl.ds(o, 16)] + b_v[pl.ds(o, 16)]
            return 0

        lax.fori_loop(0, GPAD // 16, fill, 0)
        first = rv[0:16]
        rv[0:16] = jnp.where(lane == 0, fc * 0.01, first)
        for k in range(NT // 16):
            tk = tag_v[pl.ds(16 * k, 16)]
            idx = tk + 1
            gk = plsc.load_gather(g_v, [idx])
            bk = plsc.load_gather(b_v, [idx])
            ck = c_v[pl.ds(16 * k, 16)]
            val = (ck - m) * s * gk + bk
            plsc.store_scatter(rv, [idx], val)
        cp = pltpu.async_copy(rv.at[pl.ds(0, D)], y_hbm.at[b], sems[bufi])
        pending[bufi] = cp
    for p in pending:
        if p is not None:
            p.wait()


_sc_call = functools.partial(
    pl.kernel,
    out_type=jax.ShapeDtypeStruct((B, D), jnp.float32),
    mesh=plsc.VectorSubcoreMesh(core_axis_name="c", subcore_axis_name="s"),
    compiler_params=pltpu.CompilerParams(needs_layout_passes=False,
                                         use_tc_tiling_on_sc=False),
    scratch_types=[
        pltpu.VMEM((GPAD,), jnp.float32),
        pltpu.VMEM((GPAD,), jnp.float32),
        pltpu.VMEM((GPAD,), jnp.float32),
        pltpu.VMEM((GPAD,), jnp.float32),
        pltpu.VMEM((NT,), jnp.int32),
        pltpu.VMEM((NT,), jnp.float32),
        pltpu.VMEM((48,), jnp.float32),
        pltpu.SemaphoreType.DMA,
        pltpu.SemaphoreType.DMA,
        pltpu.SemaphoreType.DMA,
    ],
)(_sc_body)


def _gelu(x):
    return 0.5 * x * (1.0 + lax.erf(x * 0.7071067811865476))


def _mm1_body(y_ref, w_ref, b_ref, o_ref):
    acc = jnp.dot(y_ref[:, :], w_ref[:, :], preferred_element_type=jnp.float32)
    o_ref[:, :] = _gelu(acc + b_ref[:, :])


def _mm1_call(y, W1, b1):
    nblk = H4 // 128
    return pl.pallas_call(
        _mm1_body,
        grid=(nblk,),
        in_specs=[
            pl.BlockSpec((B, D), lambda n: (0, 0)),
            pl.BlockSpec((D, 128), lambda n: (0, n)),
            pl.BlockSpec((1, 128), lambda n: (0, n)),
        ],
        out_specs=pl.BlockSpec((B, 128), lambda n: (0, n)),
        out_shape=jax.ShapeDtypeStruct((B, H4), jnp.float32),
    )(y, W1, b1)


def _mid_body(h_ref, g_ref, bb_ref, w2_ref, b2_ref, w3_ref, b3_ref,
              w4_ref, b4_ref, enc_ref, d4_ref):
    x = h_ref[:, :]
    mu = jnp.mean(x, axis=-1, keepdims=True)
    xc = x - mu
    var = jnp.mean(xc * xc, axis=-1, keepdims=True)
    xn = xc * lax.rsqrt(var + EPS) * g_ref[:, :] + bb_ref[:, :]
    h = _gelu(jnp.dot(xn, w2_ref[:, :], preferred_element_type=jnp.float32)
              + b2_ref[:, :])
    enc = jnp.dot(h, w3_ref[:, :], preferred_element_type=jnp.float32) \
        + b3_ref[:, :]
    enc_ref[:, :] = enc
    d4_ref[:, :] = _gelu(
        jnp.dot(enc, w4_ref[:, :], preferred_element_type=jnp.float32)
        + b4_ref[:, :])


def _mid_call(h1, ln2_g, ln2_b, W2, b2, W3, b3, W4, b4):
    return pl.pallas_call(
        _mid_body,
        out_shape=(
            jax.ShapeDtypeStruct((B, 256), jnp.float32),
            jax.ShapeDtypeStruct((B, H4), jnp.float32),
        ),
    )(h1, ln2_g, ln2_b, W2, b2, W3, b3, W4, b4)


def _mm5_body(d_ref, w_ref, b_ref, o_ref):
    o_ref[:, :] = jnp.dot(d_ref[:, :], w_ref[:, :],
                          preferred_element_type=jnp.float32) + b_ref[:, :]


def _mm5_call(d4, W5, b5):
    cb = 2048
    nblk = pl.cdiv(D, cb)
    return pl.pallas_call(
        _mm5_body,
        grid=(nblk,),
        in_specs=[
            pl.BlockSpec((B, H4), lambda n: (0, 0)),
            pl.BlockSpec((H4, cb), lambda n: (0, n)),
            pl.BlockSpec((1, cb), lambda n: (0, n)),
        ],
        out_specs=pl.BlockSpec((B, cb), lambda n: (0, n)),
        out_shape=jax.ShapeDtypeStruct((B, D), jnp.float32),
    )(d4, W5, b5)


def kernel(tags, feature_counts, ln1_g, ln1_b, W1, b1, ln2_g, ln2_b,
           W2, b2, W3, b3, W4, b4, W5, b5):
    tags2 = tags.reshape(B, NT).astype(jnp.int32)
    tags_t = tags2.T

    c_t, ms8 = _stats_call(tags_t)
    c = c_t.T
    m = ms8[0]
    s = ms8[1]
    ms48 = jnp.concatenate([
        jnp.broadcast_to(m[:, None], (B, 16)),
        jnp.broadcast_to(s[:, None], (B, 16)),
        jnp.broadcast_to(feature_counts[:, None], (B, 16)),
    ], axis=1)
    zero1 = jnp.zeros((1,), jnp.float32)
    gpad = jnp.concatenate([zero1, ln1_g, jnp.zeros((GPAD - D,), jnp.float32)])
    bpad = jnp.concatenate([zero1, ln1_b, jnp.zeros((GPAD - D,), jnp.float32)])

    y = _sc_call(gpad, bpad, tags2, c, ms48)

    h1 = _mm1_call(y, W1, b1.reshape(1, H4))
    enc, d4 = _mid_call(h1, ln2_g.reshape(1, H4), ln2_b.reshape(1, H4),
                        W2, b2.reshape(1, H4), W3, b3.reshape(1, 256),
                        W4, b4.reshape(1, H4))
    dec = _mm5_call(d4, W5, b5.reshape(1, D))
    return (y, enc, dec)


# speedup vs baseline: 3.9026x; 1.2154x over previous
import functools

import jax
import jax.numpy as jnp
from jax import lax
from jax.experimental import pallas as pl
from jax.experimental.pallas import tpu as pltpu
from jax.experimental.pallas import tpu_sc as plsc

B, F, T = 256, 8, 20
NT = F * T
N = 20000
D = N + 1
H4 = 1024
EPS = 1e-5

NC, NS = 2, 16
NW = NC * NS
SPB = B // NW
ROWP = 20096


def _stats_body(tags_ref, c_ref, ms_ref, w_scr):
    t = tags_ref[:, :]
    dup = jnp.zeros((NT, B), jnp.float32)
    for d in range(1, T):
        eq = (t[d:, :] == t[:-d, :]).astype(jnp.float32)
        eq = jnp.concatenate([jnp.zeros((d, B), jnp.float32), eq], axis=0)
        valid = ((jnp.arange(NT) % T) >= d).astype(jnp.float32)
        dup = jnp.maximum(dup, eq * valid[:, None])
    w = 1.0 - dup
    w_scr[:, :] = w

    def body(k, c):
        tk = tags_ref[pl.ds(k, 1), :]
        wk = w_scr[pl.ds(k, 1), :]
        return c + jnp.where(t == tk, wk, 0.0)

    c = lax.fori_loop(0, NT, body, jnp.zeros((NT, B), jnp.float32))
    ksum = jnp.sum(w, axis=0, keepdims=True)
    s2 = jnp.sum(w * c, axis=0, keepdims=True)
    m = ksum / N
    var = s2 / N - m * m
    s = lax.rsqrt(var + EPS)
    c_ref[:, :] = c
    ms_ref[:, :] = jnp.concatenate(
        [m, s, jnp.zeros((6, B), jnp.float32)], axis=0)


def _stats_call(tags_t):
    return pl.pallas_call(
        _stats_body,
        out_shape=(
            jax.ShapeDtypeStruct((NT, B), jnp.float32),
            jax.ShapeDtypeStruct((8, B), jnp.float32),
        ),
        scratch_shapes=[pltpu.VMEM((NT, B), jnp.float32)],
    )(tags_t)


def _sc_body(gpad_hbm, bpad_hbm, tags_hbm, c_hbm, ms_hbm, y_hbm,
             g_v, b_v, rowa_v, rowb_v, tag_v, c_v, ms_v, sem_a, sem_b):
    wid = lax.axis_index("s") * NC + lax.axis_index("c")
    base = wid * SPB
    pltpu.sync_copy(gpad_hbm, g_v)
    pltpu.sync_copy(bpad_hbm, b_v)
    rows = (rowa_v, rowb_v)
    sems = (sem_a, sem_b)
    lane = jnp.arange(16, dtype=jnp.int32)
    pending = [None, None]
    for sl in range(SPB):
        bufi = sl % 2
        rv = rows[bufi]
        if pending[bufi] is not None:
            pending[bufi].wait()
        b = base + sl
        pltpu.sync_copy(tags_hbm.at[b], tag_v)
        pltpu.sync_copy(c_hbm.at[b], c_v)
        pltpu.sync_copy(ms_hbm.at[b], ms_v)
        m = ms_v[0:16]
        s = ms_v[16:32]
        fc = ms_v[32:48]
        a = -(m * s)

        def fill(j, _, rv=rv, a=a):
            o = j * 64
            for q in range(4):
                oo = o + 16 * q
                rv[pl.ds(oo, 16)] = a * g_v[pl.ds(oo, 16)] + b_v[pl.ds(oo, 16)]
            return 0

        lax.fori_loop(0, ROWP // 64, fill, 0)
        first = rv[0:16]
        rv[0:16] = jnp.where(lane == 0, fc * 0.01, first)
        for k in range(NT // 16):
            tk = tag_v[pl.ds(16 * k, 16)]
            idx = tk + 1
            gk = plsc.load_gather(g_v, [idx])
            bk = plsc.load_gather(b_v, [idx])
            ck = c_v[pl.ds(16 * k, 16)]
            val = (ck - m) * s * gk + bk
            plsc.store_scatter(rv, [idx], val)
        cp = pltpu.async_copy(rv, y_hbm.at[b], sems[bufi])
        pending[bufi] = cp
    for p in pending:
        if p is not None:
            p.wait()


_sc_call = functools.partial(
    pl.kernel,
    out_type=jax.ShapeDtypeStruct((B, ROWP), jnp.float32),
    mesh=plsc.VectorSubcoreMesh(core_axis_name="c", subcore_axis_name="s"),
    compiler_params=pltpu.CompilerParams(needs_layout_passes=False,
                                         use_tc_tiling_on_sc=True),
    scratch_types=[
        pltpu.VMEM((ROWP,), jnp.float32),
        pltpu.VMEM((ROWP,), jnp.float32),
        pltpu.VMEM((ROWP,), jnp.float32),
        pltpu.VMEM((ROWP,), jnp.float32),
        pltpu.VMEM((256,), jnp.int32),
        pltpu.VMEM((256,), jnp.float32),
        pltpu.VMEM((128,), jnp.float32),
        pltpu.SemaphoreType.DMA,
        pltpu.SemaphoreType.DMA,
    ],
)(_sc_body)


def _gelu(x):
    return 0.5 * x * (1.0 + lax.erf(x * 0.7071067811865476))


def _mm1_body(y_ref, w_ref, b_ref, o_ref):
    y = y_ref[:, pl.ds(0, D)]
    acc = jnp.dot(y, w_ref[:, :], preferred_element_type=jnp.float32)
    o_ref[:, :] = _gelu(acc + b_ref[:, :])


def _mm1_call(y2, W1, b1):
    nblk = H4 // 128
    return pl.pallas_call(
        _mm1_body,
        grid=(nblk,),
        in_specs=[
            pl.BlockSpec((B, ROWP), lambda n: (0, 0)),
            pl.BlockSpec((D, 128), lambda n: (0, n)),
            pl.BlockSpec((1, 128), lambda n: (0, n)),
        ],
        out_specs=pl.BlockSpec((B, 128), lambda n: (0, n)),
        out_shape=jax.ShapeDtypeStruct((B, H4), jnp.float32),
    )(y2, W1, b1)


def _mid_body(h_ref, g_ref, bb_ref, w2_ref, b2_ref, w3_ref, b3_ref,
              w4_ref, b4_ref, enc_ref, d4_ref):
    x = h_ref[:, :]
    mu = jnp.mean(x, axis=-1, keepdims=True)
    xc = x - mu
    var = jnp.mean(xc * xc, axis=-1, keepdims=True)
    xn = xc * lax.rsqrt(var + EPS) * g_ref[:, :] + bb_ref[:, :]
    h = _gelu(jnp.dot(xn, w2_ref[:, :], preferred_element_type=jnp.float32)
              + b2_ref[:, :])
    enc = jnp.dot(h, w3_ref[:, :], preferred_element_type=jnp.float32) \
        + b3_ref[:, :]
    enc_ref[:, :] = enc
    d4_ref[:, :] = _gelu(
        jnp.dot(enc, w4_ref[:, :], preferred_element_type=jnp.float32)
        + b4_ref[:, :])


def _mid_call(h1, ln2_g, ln2_b, W2, b2, W3, b3, W4, b4):
    return pl.pallas_call(
        _mid_body,
        out_shape=(
            jax.ShapeDtypeStruct((B, 256), jnp.float32),
            jax.ShapeDtypeStruct((B, H4), jnp.float32),
        ),
    )(h1, ln2_g, ln2_b, W2, b2, W3, b3, W4, b4)


def _mm5_body(d_ref, w_ref, b_ref, y2_ref, o_ref, oy_ref):
    o_ref[:, :] = jnp.dot(d_ref[:, :], w_ref[:, :],
                          preferred_element_type=jnp.float32) + b_ref[:, :]
    oy_ref[:, :] = y2_ref[:, :]


def _mm5_call(d4, W5, b5, y2):
    cb = 2048
    nblk = pl.cdiv(D, cb)
    return pl.pallas_call(
        _mm5_body,
        grid=(nblk,),
        in_specs=[
            pl.BlockSpec((B, H4), lambda n: (0, 0)),
            pl.BlockSpec((H4, cb), lambda n: (0, n)),
            pl.BlockSpec((1, cb), lambda n: (0, n)),
            pl.BlockSpec((B, cb), lambda n: (0, n)),
        ],
        out_specs=[
            pl.BlockSpec((B, cb), lambda n: (0, n)),
            pl.BlockSpec((B, cb), lambda n: (0, n)),
        ],
        out_shape=(
            jax.ShapeDtypeStruct((B, D), jnp.float32),
            jax.ShapeDtypeStruct((B, D), jnp.float32),
        ),
    )(d4, W5, b5, y2)


def kernel(tags, feature_counts, ln1_g, ln1_b, W1, b1, ln2_g, ln2_b,
           W2, b2, W3, b3, W4, b4, W5, b5):
    tags2 = tags.reshape(B, NT).astype(jnp.int32)
    tags_t = tags2.T

    c_t, ms8 = _stats_call(tags_t)
    m = ms8[0]
    s = ms8[1]
    tags_p = jnp.pad(tags2, ((0, 0), (0, 96)))
    c_p = jnp.pad(c_t.T, ((0, 0), (0, 96)))
    ms_p = jnp.concatenate([
        jnp.broadcast_to(m[:, None], (B, 16)),
        jnp.broadcast_to(s[:, None], (B, 16)),
        jnp.broadcast_to(feature_counts[:, None], (B, 16)),
        jnp.zeros((B, 80), jnp.float32),
    ], axis=1)
    zero1 = jnp.zeros((1,), jnp.float32)
    gpad = jnp.concatenate([zero1, ln1_g, jnp.zeros((ROWP - D,), jnp.float32)])
    bpad = jnp.concatenate([zero1, ln1_b, jnp.zeros((ROWP - D,), jnp.float32)])

    y2 = _sc_call(gpad, bpad, tags_p, c_p, ms_p)

    h1 = _mm1_call(y2, W1, b1.reshape(1, H4))
    enc, d4 = _mid_call(h1, ln2_g.reshape(1, H4), ln2_b.reshape(1, H4),
                        W2, b2.reshape(1, H4), W3, b3.reshape(1, 256),
                        W4, b4.reshape(1, H4))
    dec, y = _mm5_call(d4, W5, b5.reshape(1, D), y2)
    return (y, enc, dec)
